# baseline (speedup 1.0000x reference)
import jax
import jax.numpy as jnp
from jax import lax
from jax.experimental import pallas as pl
from jax.experimental.pallas import tpu as pltpu

B = 16384
C = 1000
SUB = 8


NCH = 4
CHB = B // NCH


def _body(c0_ref, c1_ref, c2_ref, a_any, b_any, g_any, out_ref,
          a_v, b_v, g_v, sems):
    crefs = (c2_ref, c0_ref, c1_ref)
    srcs_dsts = ((g_any, g_v), (a_any, a_v), (b_any, b_v))
    copies = []
    for k in range(NCH):
        csl = pl.ds(k * CHB, CHB)
        chunk_copies = []
        for t, (src, dst) in enumerate(srcs_dsts):
            r0 = pl.multiple_of(lax.div(crefs[t][0], SUB) * SUB, SUB)
            cp = pltpu.make_async_copy(
                src.at[pl.ds(r0, SUB), csl], dst.at[:, csl], sems.at[t, k])
            cp.start()
            chunk_copies.append(cp)
        copies.append(chunk_copies)

    def pick(ref, cref, csl):
        row = lax.rem(cref[0], SUB)
        return ref[pl.ds(row, 1), csl].reshape(CHB)

    for k in range(NCH):
        csl = pl.ds(k * CHB, CHB)
        copies[k][0].wait()
        g = pick(g_v, c2_ref, csl)
        x = jnp.maximum(1.0 - jnp.exp(g), 1e-8)
        lg = jnp.log(x)
        copies[k][1].wait()
        copies[k][2].wait()
        ab = pick(a_v, c0_ref, csl) + pick(b_v, c1_ref, csl)
        out_ref[csl] = jnp.maximum(ab - lg, 0.0)


@jax.jit
def _transition_loss_tc(at, bt, gt, c0, c1, c2):
    return pl.pallas_call(
        _body,
        grid_spec=pltpu.PrefetchScalarGridSpec(
            num_scalar_prefetch=3,
            grid=(),
            in_specs=[pl.BlockSpec(memory_space=pl.ANY)] * 3,
            out_specs=pl.BlockSpec(memory_space=pltpu.VMEM),
            scratch_shapes=[
                pltpu.VMEM((SUB, B), jnp.float32),
                pltpu.VMEM((SUB, B), jnp.float32),
                pltpu.VMEM((SUB, B), jnp.float32),
                pltpu.SemaphoreType.DMA((3, NCH)),
            ],
        ),
        out_shape=jax.ShapeDtypeStruct((B,), jnp.float32),
    )(c0, c1, c2, at, bt, gt)


def kernel(log_y_alpha, log_y_beta, log_y_gamma,
           alpha_index, beta_index, gamma_index):
    c0 = jnp.asarray(alpha_index, dtype=jnp.int32).reshape(1)
    c1 = jnp.asarray(beta_index, dtype=jnp.int32).reshape(1)
    c2 = jnp.asarray(gamma_index, dtype=jnp.int32).reshape(1)
    return _transition_loss_tc(
        log_y_alpha.T, log_y_beta.T, log_y_gamma.T, c0, c1, c2)


# speedup vs baseline: 1.0171x; 1.0171x over previous
import jax
import jax.numpy as jnp
from jax import lax
from jax.experimental import pallas as pl
from jax.experimental.pallas import tpu as pltpu

B = 16384
C = 1000
SUB = 8


NCH = 1
CHB = B // NCH


def _body(c0_ref, c1_ref, c2_ref, a_any, b_any, g_any, out_ref,
          a_v, b_v, g_v, sems):
    crefs = (c2_ref, c0_ref, c1_ref)
    srcs_dsts = ((g_any, g_v), (a_any, a_v), (b_any, b_v))
    copies = []
    for k in range(NCH):
        csl = pl.ds(k * CHB, CHB)
        chunk_copies = []
        for t, (src, dst) in enumerate(srcs_dsts):
            r0 = pl.multiple_of(lax.div(crefs[t][0], SUB) * SUB, SUB)
            cp = pltpu.make_async_copy(
                src.at[pl.ds(r0, SUB), csl], dst.at[:, csl], sems.at[t, k])
            cp.start()
            chunk_copies.append(cp)
        copies.append(chunk_copies)

    def pick(ref, cref, csl):
        row = lax.rem(cref[0], SUB)
        return ref[pl.ds(row, 1), csl].reshape(CHB)

    for k in range(NCH):
        csl = pl.ds(k * CHB, CHB)
        copies[k][0].wait()
        g = pick(g_v, c2_ref, csl)
        x = jnp.maximum(1.0 - jnp.exp(g), 1e-8)
        lg = jnp.log(x)
        copies[k][1].wait()
        copies[k][2].wait()
        ab = pick(a_v, c0_ref, csl) + pick(b_v, c1_ref, csl)
        out_ref[csl] = jnp.maximum(ab - lg, 0.0)


@jax.jit
def _transition_loss_tc(at, bt, gt, c0, c1, c2):
    return pl.pallas_call(
        _body,
        grid_spec=pltpu.PrefetchScalarGridSpec(
            num_scalar_prefetch=3,
            grid=(),
            in_specs=[pl.BlockSpec(memory_space=pl.ANY)] * 3,
            out_specs=pl.BlockSpec(memory_space=pltpu.VMEM),
            scratch_shapes=[
                pltpu.VMEM((SUB, B), jnp.float32),
                pltpu.VMEM((SUB, B), jnp.float32),
                pltpu.VMEM((SUB, B), jnp.float32),
                pltpu.SemaphoreType.DMA((3, NCH)),
            ],
        ),
        out_shape=jax.ShapeDtypeStruct((B,), jnp.float32),
    )(c0, c1, c2, at, bt, gt)


def kernel(log_y_alpha, log_y_beta, log_y_gamma,
           alpha_index, beta_index, gamma_index):
    c0 = jnp.asarray(alpha_index, dtype=jnp.int32).reshape(1)
    c1 = jnp.asarray(beta_index, dtype=jnp.int32).reshape(1)
    c2 = jnp.asarray(gamma_index, dtype=jnp.int32).reshape(1)
    return _transition_loss_tc(
        log_y_alpha.T, log_y_beta.T, log_y_gamma.T, c0, c1, c2)
